# Initial kernel scaffold; baseline (speedup 1.0000x reference)
import jax
import jax.numpy as jnp
from jax.experimental import pallas as pl


def kernel(X, edge_index1, edge_index2, edge_index3, edge_index4, edge_index5, batch, Wr0, root0, b0, Wr1, root1, b1, Wr2, root2, b2, W1, bl1, W2, bl2, W3, bl3):
    raise NotImplementedError("write your pallas kernel here")


# speedup vs baseline: 1.5025x; 1.5025x over previous
import functools

import jax
import jax.numpy as jnp
from jax import lax
from jax.experimental import pallas as pl
from jax.experimental.pallas import tpu as pltpu
from jax.experimental.pallas import tpu_sc as plsc

N = 10000
E = 64000
R = 5
FIN = 162
H = 128
G = 128

NC = 2
NS = 16
NW = NC * NS
NPAD = 10240
EPAD = 65536
EW = EPAD // NW
CH = 128
NCH = EW // CH
TRASH = N
FB = FIN - H
ROWS_S = NPAD // NS
ZROWS = 160
BN = 512


def _make_sc_agg():
    mesh = plsc.VectorSubcoreMesh(core_axis_name="c", subcore_axis_name="s",
                                  num_cores=NC, num_subcores=NS)

    @functools.partial(
        pl.kernel,
        out_type=jax.ShapeDtypeStruct((R, NC, NPAD, H), jnp.float32),
        mesh=mesh,
        scratch_types=[
            pltpu.VMEM((NCH, CH), jnp.int32),
            pltpu.VMEM((NCH, CH), jnp.int32),
            pltpu.VMEM((CH, H), jnp.float32),
            pltpu.VMEM((ZROWS, H), jnp.float32),
            pltpu.VMEM_SHARED((NPAD, H), jnp.float32),
            pltpu.SemaphoreType.DMA,
        ],
    )
    def sc_agg(x_hbm, src_hbm, dst_hbm, zer_hbm, out_hbm,
               src_v, dst_v, rows_v, zbuf, acc_sh, sem):
        c = lax.axis_index("c")
        s = lax.axis_index("s")
        wid = s * NC + c
        pltpu.sync_copy(zer_hbm, zbuf)
        for r in range(R):
            pltpu.sync_copy(src_hbm.at[r, wid], src_v)
            pltpu.sync_copy(dst_hbm.at[r, wid], dst_v)
            for z in range(ROWS_S // ZROWS):
                pltpu.sync_copy(
                    zbuf, acc_sh.at[pl.ds(s * ROWS_S + z * ZROWS, ZROWS)])
            plsc.subcore_barrier()

            def chunk(j, carry):
                pltpu.async_copy(x_hbm.at[src_v.at[j]], rows_v, sem).wait()
                pltpu.sync_copy(rows_v, acc_sh.at[dst_v.at[j]], add=True)
                return carry

            lax.fori_loop(0, NCH, chunk, 0)
            plsc.subcore_barrier()
            pltpu.sync_copy(acc_sh.at[pl.ds(s * ROWS_S, ROWS_S)],
                            out_hbm.at[r, c, pl.ds(s * ROWS_S, ROWS_S)])

    return sc_agg


@functools.lru_cache(maxsize=None)
def _get_sc_agg():
    return _make_sc_agg()


def _dense0_body(xa_ref, xb_ref, pa_ref, pb_ref, wa_ref, wb_ref,
                 ra_ref, rb_ref, b_ref, h_ref, ic_ref):
    acc = jnp.dot(xa_ref[...], ra_ref[...], preferred_element_type=jnp.float32)
    acc = acc + jnp.dot(xb_ref[...], rb_ref[...],
                        preferred_element_type=jnp.float32)
    acc = acc + b_ref[...]
    invs = []
    for r in range(R):
        agga = pa_ref[r, 0] + pa_ref[r, 1]
        aggb = pb_ref[r, 0] + pb_ref[r, 1]
        cnt = aggb[:, H - 1]
        inv = 1.0 / jnp.maximum(cnt, 1.0)
        invs.append(inv[:, None])
        acc = acc + jnp.dot(agga * inv[:, None], wa_ref[r],
                            preferred_element_type=jnp.float32)
        acc = acc + jnp.dot(aggb * inv[:, None], wb_ref[r],
                            preferred_element_type=jnp.float32)
    h_ref[...] = jnp.maximum(acc, 0.0)
    ic = jnp.concatenate(invs + [jnp.ones((BN, 3), jnp.float32)], axis=1)
    ic_ref[...] = ic


def _dense0(xa, xb, pa, pb, wa, wb, ra, rb, b):
    grid = (NPAD // BN,)
    return pl.pallas_call(
        _dense0_body,
        grid=grid,
        in_specs=[
            pl.BlockSpec((BN, H), lambda i: (i, 0)),
            pl.BlockSpec((BN, H), lambda i: (i, 0)),
            pl.BlockSpec((R, NC, BN, H), lambda i: (0, 0, i, 0)),
            pl.BlockSpec((R, NC, BN, H), lambda i: (0, 0, i, 0)),
            pl.BlockSpec((R, H, H), lambda i: (0, 0, 0)),
            pl.BlockSpec((R, H, H), lambda i: (0, 0, 0)),
            pl.BlockSpec((H, H), lambda i: (0, 0)),
            pl.BlockSpec((H, H), lambda i: (0, 0)),
            pl.BlockSpec((1, H), lambda i: (0, 0)),
        ],
        out_specs=[
            pl.BlockSpec((BN, H), lambda i: (i, 0)),
            pl.BlockSpec((BN, 8), lambda i: (i, 0)),
        ],
        out_shape=[
            jax.ShapeDtypeStruct((NPAD, H), jnp.float32),
            jax.ShapeDtypeStruct((NPAD, 8), jnp.float32),
        ],
    )(xa, xb, pa, pb, wa, wb, ra, rb, b)


def _dense_body(x_ref, p_ref, ic_ref, w_ref, rt_ref, b_ref, h_ref):
    acc = jnp.dot(x_ref[...], rt_ref[...], preferred_element_type=jnp.float32)
    acc = acc + b_ref[...]
    ic = ic_ref[...]
    for r in range(R):
        agg = p_ref[r, 0] + p_ref[r, 1]
        acc = acc + jnp.dot(agg * ic[:, r][:, None], w_ref[r],
                            preferred_element_type=jnp.float32)
    h_ref[...] = jnp.maximum(acc, 0.0)


def _dense(h, parts, invc, wr, root, b):
    grid = (NPAD // BN,)
    return pl.pallas_call(
        _dense_body,
        grid=grid,
        in_specs=[
            pl.BlockSpec((BN, H), lambda i: (i, 0)),
            pl.BlockSpec((R, NC, BN, H), lambda i: (0, 0, i, 0)),
            pl.BlockSpec((BN, 8), lambda i: (i, 0)),
            pl.BlockSpec((R, H, H), lambda i: (0, 0, 0)),
            pl.BlockSpec((H, H), lambda i: (0, 0)),
            pl.BlockSpec((1, H), lambda i: (0, 0)),
        ],
        out_specs=pl.BlockSpec((BN, H), lambda i: (i, 0)),
        out_shape=jax.ShapeDtypeStruct((NPAD, H), jnp.float32),
    )(h, parts, invc, wr, root, b)


def _head_body(h_ref, bt_ref, w1_ref, b1_ref, w2_ref, b2_ref, w3_ref, b3_ref,
               out_ref, acc_ref):
    i = pl.program_id(0)

    @pl.when(i == 0)
    def _():
        acc_ref[...] = jnp.zeros((G, H), jnp.float32)

    bids = bt_ref[...]
    onehot = (bids == lax.broadcasted_iota(jnp.int32, (BN, G), 1))
    onehot = onehot.astype(jnp.float32)
    acc_ref[...] += lax.dot_general(
        onehot, h_ref[...], (((0,), (0,)), ((), ())),
        preferred_element_type=jnp.float32)

    @pl.when(i == pl.num_programs(0) - 1)
    def _():
        ro = acc_ref[...]
        z = jnp.maximum(jnp.dot(ro, w1_ref[...],
                                preferred_element_type=jnp.float32)
                        + b1_ref[...], 0.0)
        z = jnp.maximum(jnp.dot(z, w2_ref[...],
                                preferred_element_type=jnp.float32)
                        + b2_ref[...], 0.0)
        out_ref[...] = jnp.dot(z, w3_ref[...],
                               preferred_element_type=jnp.float32) + b3_ref[...]


def _head(h3, batch_p, w1, b1, w2, b2, w3, b3):
    grid = (NPAD // BN,)
    return pl.pallas_call(
        _head_body,
        grid=grid,
        in_specs=[
            pl.BlockSpec((BN, H), lambda i: (i, 0)),
            pl.BlockSpec((BN, 1), lambda i: (i, 0)),
            pl.BlockSpec((H, H), lambda i: (0, 0)),
            pl.BlockSpec((1, H), lambda i: (0, 0)),
            pl.BlockSpec((H, H), lambda i: (0, 0)),
            pl.BlockSpec((1, H), lambda i: (0, 0)),
            pl.BlockSpec((H, 1), lambda i: (0, 0)),
            pl.BlockSpec((1, 1), lambda i: (0, 0)),
        ],
        out_specs=pl.BlockSpec((G, 1), lambda i: (0, 0)),
        out_shape=jax.ShapeDtypeStruct((G, 1), jnp.float32),
        scratch_shapes=[pltpu.VMEM((G, H), jnp.float32)],
    )(h3, batch_p, w1, b1, w2, b2, w3, b3)


def kernel(X, edge_index1, edge_index2, edge_index3, edge_index4, edge_index5,
           batch, Wr0, root0, b0, Wr1, root1, b1, Wr2, root2, b2,
           W1, bl1, W2, bl2, W3, bl3):
    f32 = jnp.float32
    zrows = jnp.zeros((NPAD - N, H), f32)
    xa = jnp.concatenate([X[:, :H], zrows], axis=0)
    xb = jnp.concatenate([X[:, H:], jnp.zeros((N, H - FB - 1), f32),
                          jnp.ones((N, 1), f32)], axis=1)
    xb = jnp.concatenate([xb, zrows], axis=0)

    eis = jnp.stack([edge_index1, edge_index2, edge_index3,
                     edge_index4, edge_index5])
    src = jnp.concatenate(
        [eis[:, 0, :], jnp.zeros((R, EPAD - E), jnp.int32)], axis=1)
    dst = jnp.concatenate(
        [eis[:, 1, :], jnp.full((R, EPAD - E), TRASH, jnp.int32)], axis=1)
    src = src.reshape(R, NW, NCH, CH)
    dst = dst.reshape(R, NW, NCH, CH)

    wa = Wr0[:, :H, :]
    wb = jnp.concatenate([Wr0[:, H:, :],
                          jnp.zeros((R, H - FB, H), f32)], axis=1)
    ra = root0[:H, :]
    rb = jnp.concatenate([root0[H:, :], jnp.zeros((H - FB, H), f32)], axis=0)
    batch_p = jnp.concatenate(
        [batch, jnp.full((NPAD - N,), G, jnp.int32)]).reshape(NPAD, 1)
    zer = jnp.zeros((ZROWS, H), f32)

    sc_agg = _get_sc_agg()
    pa = sc_agg(xa, src, dst, zer)
    pb = sc_agg(xb, src, dst, zer)
    h1, invc = _dense0(xa, xb, pa, pb, wa, wb, ra, rb, b0.reshape(1, H))
    parts1 = sc_agg(h1, src, dst, zer)
    h2 = _dense(h1, parts1, invc, Wr1, root1, b1.reshape(1, H))
    parts2 = sc_agg(h2, src, dst, zer)
    h3 = _dense(h2, parts2, invc, Wr2, root2, b2.reshape(1, H))
    return _head(h3, batch_p, W1, bl1.reshape(1, H), W2, bl2.reshape(1, H),
                 W3, bl3.reshape(1, 1))


# speedup vs baseline: 1.7135x; 1.1405x over previous
import functools

import jax
import jax.numpy as jnp
from jax import lax
from jax.experimental import pallas as pl
from jax.experimental.pallas import tpu as pltpu
from jax.experimental.pallas import tpu_sc as plsc

N = 10000
E = 64000
R = 5
FIN = 162
H = 128
G = 128

NC = 2
NS = 16
NW = NC * NS
NPAD = 10240
EPAD = 65536
EW = EPAD // NW
CH = 128
NCH = EW // CH
TRASH = N
FB = FIN - H
ROWS_S = NPAD // NS
ZROWS = 64
BN = 512


def _make_sc_agg():
    mesh = plsc.VectorSubcoreMesh(core_axis_name="c", subcore_axis_name="s",
                                  num_cores=NC, num_subcores=NS)

    NBUF = 2

    @functools.partial(
        pl.kernel,
        out_type=jax.ShapeDtypeStruct((R, NC, NPAD, H), jnp.float32),
        mesh=mesh,
        scratch_types=[
            pltpu.VMEM((NCH, CH), jnp.int32),
            pltpu.VMEM((NCH, CH), jnp.int32),
            [pltpu.VMEM((CH, H), jnp.float32) for _ in range(NBUF)],
            pltpu.VMEM((ZROWS, H), jnp.float32),
            pltpu.VMEM_SHARED((NPAD, H), jnp.float32),
            [pltpu.SemaphoreType.DMA for _ in range(NBUF)],
            [pltpu.SemaphoreType.DMA for _ in range(NBUF)],
        ],
    )
    def sc_agg(x_hbm, src_hbm, dst_hbm, zer_hbm, out_hbm,
               src_v, dst_v, rows, zbuf, acc_sh, gsem, ssem):
        c = lax.axis_index("c")
        s = lax.axis_index("s")
        wid = s * NC + c
        pltpu.sync_copy(zer_hbm, zbuf)
        for r in range(R):
            pltpu.sync_copy(src_hbm.at[r, wid], src_v)
            pltpu.sync_copy(dst_hbm.at[r, wid], dst_v)
            for z in range(ROWS_S // ZROWS):
                pltpu.sync_copy(
                    zbuf, acc_sh.at[pl.ds(s * ROWS_S + z * ZROWS, ZROWS)])
            plsc.subcore_barrier()

            gcp = [None] * NBUF
            scp = [None] * NBUF
            for j in range(NCH + 1):
                if j < NCH:
                    b = j % NBUF
                    if scp[b] is not None:
                        scp[b].wait()
                    gcp[b] = pltpu.async_copy(
                        x_hbm.at[src_v.at[j]], rows[b], gsem[b])
                if j >= 1:
                    pb = (j - 1) % NBUF
                    gcp[pb].wait()
                    scp[pb] = pltpu.async_copy(
                        rows[pb], acc_sh.at[dst_v.at[j - 1]], ssem[pb],
                        add=True)
            for b in range(NBUF):
                if scp[b] is not None:
                    scp[b].wait()
            plsc.subcore_barrier()
            pltpu.sync_copy(acc_sh.at[pl.ds(s * ROWS_S, ROWS_S)],
                            out_hbm.at[r, c, pl.ds(s * ROWS_S, ROWS_S)])

    return sc_agg


@functools.lru_cache(maxsize=None)
def _get_sc_agg():
    return _make_sc_agg()


def _dense0_body(xa_ref, xb_ref, pa_ref, pb_ref, wa_ref, wb_ref,
                 ra_ref, rb_ref, b_ref, h_ref, ic_ref):
    acc = jnp.dot(xa_ref[...], ra_ref[...], preferred_element_type=jnp.float32)
    acc = acc + jnp.dot(xb_ref[...], rb_ref[...],
                        preferred_element_type=jnp.float32)
    acc = acc + b_ref[...]
    invs = []
    for r in range(R):
        agga = pa_ref[r, 0] + pa_ref[r, 1]
        aggb = pb_ref[r, 0] + pb_ref[r, 1]
        cnt = aggb[:, H - 1]
        inv = 1.0 / jnp.maximum(cnt, 1.0)
        invs.append(inv[:, None])
        acc = acc + jnp.dot(agga * inv[:, None], wa_ref[r],
                            preferred_element_type=jnp.float32)
        acc = acc + jnp.dot(aggb * inv[:, None], wb_ref[r],
                            preferred_element_type=jnp.float32)
    h_ref[...] = jnp.maximum(acc, 0.0)
    ic = jnp.concatenate(invs + [jnp.ones((BN, 3), jnp.float32)], axis=1)
    ic_ref[...] = ic


def _dense0(xa, xb, pa, pb, wa, wb, ra, rb, b):
    grid = (NPAD // BN,)
    return pl.pallas_call(
        _dense0_body,
        grid=grid,
        in_specs=[
            pl.BlockSpec((BN, H), lambda i: (i, 0)),
            pl.BlockSpec((BN, H), lambda i: (i, 0)),
            pl.BlockSpec((R, NC, BN, H), lambda i: (0, 0, i, 0)),
            pl.BlockSpec((R, NC, BN, H), lambda i: (0, 0, i, 0)),
            pl.BlockSpec((R, H, H), lambda i: (0, 0, 0)),
            pl.BlockSpec((R, H, H), lambda i: (0, 0, 0)),
            pl.BlockSpec((H, H), lambda i: (0, 0)),
            pl.BlockSpec((H, H), lambda i: (0, 0)),
            pl.BlockSpec((1, H), lambda i: (0, 0)),
        ],
        out_specs=[
            pl.BlockSpec((BN, H), lambda i: (i, 0)),
            pl.BlockSpec((BN, 8), lambda i: (i, 0)),
        ],
        out_shape=[
            jax.ShapeDtypeStruct((NPAD, H), jnp.float32),
            jax.ShapeDtypeStruct((NPAD, 8), jnp.float32),
        ],
    )(xa, xb, pa, pb, wa, wb, ra, rb, b)


def _dense_body(x_ref, p_ref, ic_ref, w_ref, rt_ref, b_ref, h_ref):
    acc = jnp.dot(x_ref[...], rt_ref[...], preferred_element_type=jnp.float32)
    acc = acc + b_ref[...]
    ic = ic_ref[...]
    for r in range(R):
        agg = p_ref[r, 0] + p_ref[r, 1]
        acc = acc + jnp.dot(agg * ic[:, r][:, None], w_ref[r],
                            preferred_element_type=jnp.float32)
    h_ref[...] = jnp.maximum(acc, 0.0)


def _dense(h, parts, invc, wr, root, b):
    grid = (NPAD // BN,)
    return pl.pallas_call(
        _dense_body,
        grid=grid,
        in_specs=[
            pl.BlockSpec((BN, H), lambda i: (i, 0)),
            pl.BlockSpec((R, NC, BN, H), lambda i: (0, 0, i, 0)),
            pl.BlockSpec((BN, 8), lambda i: (i, 0)),
            pl.BlockSpec((R, H, H), lambda i: (0, 0, 0)),
            pl.BlockSpec((H, H), lambda i: (0, 0)),
            pl.BlockSpec((1, H), lambda i: (0, 0)),
        ],
        out_specs=pl.BlockSpec((BN, H), lambda i: (i, 0)),
        out_shape=jax.ShapeDtypeStruct((NPAD, H), jnp.float32),
    )(h, parts, invc, wr, root, b)


def _head_body(h_ref, bt_ref, w1_ref, b1_ref, w2_ref, b2_ref, w3_ref, b3_ref,
               out_ref, acc_ref):
    i = pl.program_id(0)

    @pl.when(i == 0)
    def _():
        acc_ref[...] = jnp.zeros((G, H), jnp.float32)

    bids = bt_ref[...]
    onehot = (bids == lax.broadcasted_iota(jnp.int32, (BN, G), 1))
    onehot = onehot.astype(jnp.float32)
    acc_ref[...] += lax.dot_general(
        onehot, h_ref[...], (((0,), (0,)), ((), ())),
        preferred_element_type=jnp.float32)

    @pl.when(i == pl.num_programs(0) - 1)
    def _():
        ro = acc_ref[...]
        z = jnp.maximum(jnp.dot(ro, w1_ref[...],
                                preferred_element_type=jnp.float32)
                        + b1_ref[...], 0.0)
        z = jnp.maximum(jnp.dot(z, w2_ref[...],
                                preferred_element_type=jnp.float32)
                        + b2_ref[...], 0.0)
        out_ref[...] = jnp.dot(z, w3_ref[...],
                               preferred_element_type=jnp.float32) + b3_ref[...]


def _head(h3, batch_p, w1, b1, w2, b2, w3, b3):
    grid = (NPAD // BN,)
    return pl.pallas_call(
        _head_body,
        grid=grid,
        in_specs=[
            pl.BlockSpec((BN, H), lambda i: (i, 0)),
            pl.BlockSpec((BN, 1), lambda i: (i, 0)),
            pl.BlockSpec((H, H), lambda i: (0, 0)),
            pl.BlockSpec((1, H), lambda i: (0, 0)),
            pl.BlockSpec((H, H), lambda i: (0, 0)),
            pl.BlockSpec((1, H), lambda i: (0, 0)),
            pl.BlockSpec((H, 1), lambda i: (0, 0)),
            pl.BlockSpec((1, 1), lambda i: (0, 0)),
        ],
        out_specs=pl.BlockSpec((G, 1), lambda i: (0, 0)),
        out_shape=jax.ShapeDtypeStruct((G, 1), jnp.float32),
        scratch_shapes=[pltpu.VMEM((G, H), jnp.float32)],
    )(h3, batch_p, w1, b1, w2, b2, w3, b3)


def kernel(X, edge_index1, edge_index2, edge_index3, edge_index4, edge_index5,
           batch, Wr0, root0, b0, Wr1, root1, b1, Wr2, root2, b2,
           W1, bl1, W2, bl2, W3, bl3):
    f32 = jnp.float32
    zrows = jnp.zeros((NPAD - N, H), f32)
    xa = jnp.concatenate([X[:, :H], zrows], axis=0)
    xb = jnp.concatenate([X[:, H:], jnp.zeros((N, H - FB - 1), f32),
                          jnp.ones((N, 1), f32)], axis=1)
    xb = jnp.concatenate([xb, zrows], axis=0)

    eis = jnp.stack([edge_index1, edge_index2, edge_index3,
                     edge_index4, edge_index5])
    src = jnp.concatenate(
        [eis[:, 0, :], jnp.zeros((R, EPAD - E), jnp.int32)], axis=1)
    dst = jnp.concatenate(
        [eis[:, 1, :], jnp.full((R, EPAD - E), TRASH, jnp.int32)], axis=1)
    src = src.reshape(R, NW, NCH, CH)
    dst = dst.reshape(R, NW, NCH, CH)

    wa = Wr0[:, :H, :]
    wb = jnp.concatenate([Wr0[:, H:, :],
                          jnp.zeros((R, H - FB, H), f32)], axis=1)
    ra = root0[:H, :]
    rb = jnp.concatenate([root0[H:, :], jnp.zeros((H - FB, H), f32)], axis=0)
    batch_p = jnp.concatenate(
        [batch, jnp.full((NPAD - N,), G, jnp.int32)]).reshape(NPAD, 1)
    zer = jnp.zeros((ZROWS, H), f32)

    sc_agg = _get_sc_agg()
    pa = sc_agg(xa, src, dst, zer)
    pb = sc_agg(xb, src, dst, zer)
    h1, invc = _dense0(xa, xb, pa, pb, wa, wb, ra, rb, b0.reshape(1, H))
    parts1 = sc_agg(h1, src, dst, zer)
    h2 = _dense(h1, parts1, invc, Wr1, root1, b1.reshape(1, H))
    parts2 = sc_agg(h2, src, dst, zer)
    h3 = _dense(h2, parts2, invc, Wr2, root2, b2.reshape(1, H))
    return _head(h3, batch_p, W1, bl1.reshape(1, H), W2, bl2.reshape(1, H),
                 W3, bl3.reshape(1, 1))


# speedup vs baseline: 1.8019x; 1.0516x over previous
import functools

import jax
import jax.numpy as jnp
from jax import lax
from jax.experimental import pallas as pl
from jax.experimental.pallas import tpu as pltpu
from jax.experimental.pallas import tpu_sc as plsc

N = 10000
E = 64000
R = 5
FIN = 162
H = 128
G = 128

NC = 2
NS = 16
NW = NC * NS
NPAD = 10240
EPAD = 65536
EW = EPAD // NW
CH = 128
NCH = EW // CH
TRASH = N
FB = FIN - H
ROWS_S = NPAD // NS
ZROWS = 32
BN = 512


def _make_sc_agg():
    mesh = plsc.VectorSubcoreMesh(core_axis_name="c", subcore_axis_name="s",
                                  num_cores=NC, num_subcores=NS)

    NBUF = 2

    @functools.partial(
        pl.kernel,
        out_type=jax.ShapeDtypeStruct((R, NC, NPAD, H), jnp.float32),
        mesh=mesh,
        scratch_types=[
            [pltpu.VMEM((NCH, CH), jnp.int32) for _ in range(2)],
            [pltpu.VMEM((NCH, CH), jnp.int32) for _ in range(2)],
            [pltpu.VMEM((CH, H), jnp.float32) for _ in range(NBUF)],
            pltpu.VMEM((ZROWS, H), jnp.float32),
            pltpu.VMEM_SHARED((NPAD, H), jnp.float32),
            [pltpu.SemaphoreType.DMA for _ in range(NBUF)],
            [pltpu.SemaphoreType.DMA for _ in range(NBUF)],
            pltpu.SemaphoreType.DMA,
        ],
    )
    def sc_agg(x_hbm, src_hbm, dst_hbm, zer_hbm, out_hbm,
               src_v, dst_v, rows, zbuf, acc_sh, gsem, ssem, msem):
        c = lax.axis_index("c")
        s = lax.axis_index("s")
        wid = s * NC + c
        pltpu.sync_copy(zer_hbm, zbuf)
        pltpu.sync_copy(src_hbm.at[0, wid], src_v[0])
        pltpu.sync_copy(dst_hbm.at[0, wid], dst_v[0])
        for z in range(ROWS_S // ZROWS):
            pltpu.sync_copy(
                zbuf, acc_sh.at[pl.ds(s * ROWS_S + z * ZROWS, ZROWS)])
        plsc.subcore_barrier()

        for r in range(R):
            sv, dv = src_v[r % 2], dst_v[r % 2]
            gcp = [None] * NBUF
            scp = [None] * NBUF
            for j in range(NCH + 1):
                if j < NCH:
                    b = j % NBUF
                    if scp[b] is not None:
                        scp[b].wait()
                    gcp[b] = pltpu.async_copy(
                        x_hbm.at[sv.at[j]], rows[b], gsem[b])
                if j >= 1:
                    pb = (j - 1) % NBUF
                    gcp[pb].wait()
                    scp[pb] = pltpu.async_copy(
                        rows[pb], acc_sh.at[dv.at[j - 1]], ssem[pb],
                        add=True)
            for b in range(NBUF):
                if scp[b] is not None:
                    scp[b].wait()
            plsc.subcore_barrier()
            pltpu.sync_copy(acc_sh.at[pl.ds(s * ROWS_S, ROWS_S)],
                            out_hbm.at[r, c, pl.ds(s * ROWS_S, ROWS_S)])
            if r + 1 < R:
                pltpu.sync_copy(src_hbm.at[r + 1, wid], src_v[(r + 1) % 2])
                pltpu.sync_copy(dst_hbm.at[r + 1, wid], dst_v[(r + 1) % 2])
                plsc.subcore_barrier()

    return sc_agg


@functools.lru_cache(maxsize=None)
def _get_sc_agg():
    return _make_sc_agg()


def _dense0_body(xa_ref, xb_ref, pa_ref, pb_ref, wa_ref, wb_ref,
                 ra_ref, rb_ref, b_ref, h_ref, ic_ref):
    acc = jnp.dot(xa_ref[...], ra_ref[...], preferred_element_type=jnp.float32)
    acc = acc + jnp.dot(xb_ref[...], rb_ref[...],
                        preferred_element_type=jnp.float32)
    acc = acc + b_ref[...]
    invs = []
    sa_prev = jnp.zeros((BN, H), jnp.float32)
    sb_prev = jnp.zeros((BN, H), jnp.float32)
    for r in range(R):
        sa_cur = pa_ref[r, 0] + pa_ref[r, 1]
        sb_cur = pb_ref[r, 0] + pb_ref[r, 1]
        agga = sa_cur - sa_prev
        aggb = sb_cur - sb_prev
        sa_prev, sb_prev = sa_cur, sb_cur
        cnt = aggb[:, H - 1]
        inv = 1.0 / jnp.maximum(cnt, 1.0)
        invs.append(inv[:, None])
        acc = acc + jnp.dot(agga * inv[:, None], wa_ref[r],
                            preferred_element_type=jnp.float32)
        acc = acc + jnp.dot(aggb * inv[:, None], wb_ref[r],
                            preferred_element_type=jnp.float32)
    h_ref[...] = jnp.maximum(acc, 0.0)
    ic = jnp.concatenate(invs + [jnp.ones((BN, 3), jnp.float32)], axis=1)
    ic_ref[...] = ic


def _dense0(xa, xb, pa, pb, wa, wb, ra, rb, b):
    grid = (NPAD // BN,)
    return pl.pallas_call(
        _dense0_body,
        grid=grid,
        in_specs=[
            pl.BlockSpec((BN, H), lambda i: (i, 0)),
            pl.BlockSpec((BN, H), lambda i: (i, 0)),
            pl.BlockSpec((R, NC, BN, H), lambda i: (0, 0, i, 0)),
            pl.BlockSpec((R, NC, BN, H), lambda i: (0, 0, i, 0)),
            pl.BlockSpec((R, H, H), lambda i: (0, 0, 0)),
            pl.BlockSpec((R, H, H), lambda i: (0, 0, 0)),
            pl.BlockSpec((H, H), lambda i: (0, 0)),
            pl.BlockSpec((H, H), lambda i: (0, 0)),
            pl.BlockSpec((1, H), lambda i: (0, 0)),
        ],
        out_specs=[
            pl.BlockSpec((BN, H), lambda i: (i, 0)),
            pl.BlockSpec((BN, 8), lambda i: (i, 0)),
        ],
        out_shape=[
            jax.ShapeDtypeStruct((NPAD, H), jnp.float32),
            jax.ShapeDtypeStruct((NPAD, 8), jnp.float32),
        ],
    )(xa, xb, pa, pb, wa, wb, ra, rb, b)


def _dense_body(x_ref, p_ref, ic_ref, w_ref, rt_ref, b_ref, h_ref):
    acc = jnp.dot(x_ref[...], rt_ref[...], preferred_element_type=jnp.float32)
    acc = acc + b_ref[...]
    ic = ic_ref[...]
    s_prev = jnp.zeros((BN, H), jnp.float32)
    for r in range(R):
        s_cur = p_ref[r, 0] + p_ref[r, 1]
        agg = s_cur - s_prev
        s_prev = s_cur
        acc = acc + jnp.dot(agg * ic[:, r][:, None], w_ref[r],
                            preferred_element_type=jnp.float32)
    h_ref[...] = jnp.maximum(acc, 0.0)


def _dense(h, parts, invc, wr, root, b):
    grid = (NPAD // BN,)
    return pl.pallas_call(
        _dense_body,
        grid=grid,
        in_specs=[
            pl.BlockSpec((BN, H), lambda i: (i, 0)),
            pl.BlockSpec((R, NC, BN, H), lambda i: (0, 0, i, 0)),
            pl.BlockSpec((BN, 8), lambda i: (i, 0)),
            pl.BlockSpec((R, H, H), lambda i: (0, 0, 0)),
            pl.BlockSpec((H, H), lambda i: (0, 0)),
            pl.BlockSpec((1, H), lambda i: (0, 0)),
        ],
        out_specs=pl.BlockSpec((BN, H), lambda i: (i, 0)),
        out_shape=jax.ShapeDtypeStruct((NPAD, H), jnp.float32),
    )(h, parts, invc, wr, root, b)


def _head_body(h_ref, bt_ref, w1_ref, b1_ref, w2_ref, b2_ref, w3_ref, b3_ref,
               out_ref, acc_ref):
    i = pl.program_id(0)

    @pl.when(i == 0)
    def _():
        acc_ref[...] = jnp.zeros((G, H), jnp.float32)

    bids = bt_ref[...]
    onehot = (bids == lax.broadcasted_iota(jnp.int32, (BN, G), 1))
    onehot = onehot.astype(jnp.float32)
    acc_ref[...] += lax.dot_general(
        onehot, h_ref[...], (((0,), (0,)), ((), ())),
        preferred_element_type=jnp.float32)

    @pl.when(i == pl.num_programs(0) - 1)
    def _():
        ro = acc_ref[...]
        z = jnp.maximum(jnp.dot(ro, w1_ref[...],
                                preferred_element_type=jnp.float32)
                        + b1_ref[...], 0.0)
        z = jnp.maximum(jnp.dot(z, w2_ref[...],
                                preferred_element_type=jnp.float32)
                        + b2_ref[...], 0.0)
        out_ref[...] = jnp.dot(z, w3_ref[...],
                               preferred_element_type=jnp.float32) + b3_ref[...]


def _head(h3, batch_p, w1, b1, w2, b2, w3, b3):
    grid = (NPAD // BN,)
    return pl.pallas_call(
        _head_body,
        grid=grid,
        in_specs=[
            pl.BlockSpec((BN, H), lambda i: (i, 0)),
            pl.BlockSpec((BN, 1), lambda i: (i, 0)),
            pl.BlockSpec((H, H), lambda i: (0, 0)),
            pl.BlockSpec((1, H), lambda i: (0, 0)),
            pl.BlockSpec((H, H), lambda i: (0, 0)),
            pl.BlockSpec((1, H), lambda i: (0, 0)),
            pl.BlockSpec((H, 1), lambda i: (0, 0)),
            pl.BlockSpec((1, 1), lambda i: (0, 0)),
        ],
        out_specs=pl.BlockSpec((G, 1), lambda i: (0, 0)),
        out_shape=jax.ShapeDtypeStruct((G, 1), jnp.float32),
        scratch_shapes=[pltpu.VMEM((G, H), jnp.float32)],
    )(h3, batch_p, w1, b1, w2, b2, w3, b3)


def kernel(X, edge_index1, edge_index2, edge_index3, edge_index4, edge_index5,
           batch, Wr0, root0, b0, Wr1, root1, b1, Wr2, root2, b2,
           W1, bl1, W2, bl2, W3, bl3):
    f32 = jnp.float32
    zrows = jnp.zeros((NPAD - N, H), f32)
    xa = jnp.concatenate([X[:, :H], zrows], axis=0)
    xb = jnp.concatenate([X[:, H:], jnp.zeros((N, H - FB - 1), f32),
                          jnp.ones((N, 1), f32)], axis=1)
    xb = jnp.concatenate([xb, zrows], axis=0)

    eis = jnp.stack([edge_index1, edge_index2, edge_index3,
                     edge_index4, edge_index5])
    src = jnp.concatenate(
        [eis[:, 0, :], jnp.zeros((R, EPAD - E), jnp.int32)], axis=1)
    dst = jnp.concatenate(
        [eis[:, 1, :], jnp.full((R, EPAD - E), TRASH, jnp.int32)], axis=1)
    src = src.reshape(R, NW, NCH, CH)
    dst = dst.reshape(R, NW, NCH, CH)

    wa = Wr0[:, :H, :]
    wb = jnp.concatenate([Wr0[:, H:, :],
                          jnp.zeros((R, H - FB, H), f32)], axis=1)
    ra = root0[:H, :]
    rb = jnp.concatenate([root0[H:, :], jnp.zeros((H - FB, H), f32)], axis=0)
    batch_p = jnp.concatenate(
        [batch, jnp.full((NPAD - N,), G, jnp.int32)]).reshape(NPAD, 1)
    zer = jnp.zeros((ZROWS, H), f32)

    sc_agg = _get_sc_agg()
    pa = sc_agg(xa, src, dst, zer)
    pb = sc_agg(xb, src, dst, zer)
    h1, invc = _dense0(xa, xb, pa, pb, wa, wb, ra, rb, b0.reshape(1, H))
    parts1 = sc_agg(h1, src, dst, zer)
    h2 = _dense(h1, parts1, invc, Wr1, root1, b1.reshape(1, H))
    parts2 = sc_agg(h2, src, dst, zer)
    h3 = _dense(h2, parts2, invc, Wr2, root2, b2.reshape(1, H))
    return _head(h3, batch_p, W1, bl1.reshape(1, H), W2, bl2.reshape(1, H),
                 W3, bl3.reshape(1, 1))


# speedup vs baseline: 2.1920x; 1.2165x over previous
import functools

import jax
import jax.numpy as jnp
from jax import lax
from jax.experimental import pallas as pl
from jax.experimental.pallas import tpu as pltpu
from jax.experimental.pallas import tpu_sc as plsc

N = 10000
E = 64000
R = 5
FIN = 162
H = 128
G = 128

NC = 2
NS = 16
NW = NC * NS
NPAD = 10240
EPAD = 65536
EW = EPAD // NW
CH = 128
NCH = EW // CH
TRASH = N
ROWS_S = NPAD // NS
ZROWS = 32
BN = 512
BE = 4096
NBE = EPAD // BE


def _make_sc_agg():
    mesh = plsc.VectorSubcoreMesh(core_axis_name="c", subcore_axis_name="s",
                                  num_cores=NC, num_subcores=NS)
    NBUF = 2

    @functools.partial(
        pl.kernel,
        out_type=jax.ShapeDtypeStruct((R, NC, NPAD, H), jnp.float32),
        mesh=mesh,
        scratch_types=[
            [pltpu.VMEM((NCH, CH), jnp.int32) for _ in range(2)],
            [pltpu.VMEM((NCH, CH), jnp.int32) for _ in range(2)],
            [pltpu.VMEM((CH, H), jnp.float32) for _ in range(NBUF)],
            pltpu.VMEM((ZROWS, H), jnp.float32),
            pltpu.VMEM_SHARED((NPAD, H), jnp.float32),
            [pltpu.SemaphoreType.DMA for _ in range(NBUF)],
            [pltpu.SemaphoreType.DMA for _ in range(NBUF)],
        ],
    )
    def sc_agg(y0, y1, y2, y3, y4, src_hbm, dst_hbm, zer_hbm, out_hbm,
               src_v, dst_v, rows, zbuf, acc_sh, gsem, ssem):
        ys = [y0, y1, y2, y3, y4]
        c = lax.axis_index("c")
        s = lax.axis_index("s")
        wid = s * NC + c
        pltpu.sync_copy(zer_hbm, zbuf)
        pltpu.sync_copy(src_hbm.at[0, wid], src_v[0])
        pltpu.sync_copy(dst_hbm.at[0, wid], dst_v[0])
        for z in range(ROWS_S // ZROWS):
            pltpu.sync_copy(
                zbuf, acc_sh.at[pl.ds(s * ROWS_S + z * ZROWS, ZROWS)])
        plsc.subcore_barrier()

        for r in range(R):
            sv, dv = src_v[r % 2], dst_v[r % 2]
            gcp = [None] * NBUF
            scp = [None] * NBUF
            for j in range(NCH + 1):
                if j < NCH:
                    b = j % NBUF
                    if scp[b] is not None:
                        scp[b].wait()
                    gcp[b] = pltpu.async_copy(
                        ys[r].at[sv.at[j]], rows[b], gsem[b])
                if j >= 1:
                    pb = (j - 1) % NBUF
                    gcp[pb].wait()
                    scp[pb] = pltpu.async_copy(
                        rows[pb], acc_sh.at[dv.at[j - 1]], ssem[pb],
                        add=True)
            for b in range(NBUF):
                if scp[b] is not None:
                    scp[b].wait()
            plsc.subcore_barrier()
            pltpu.sync_copy(acc_sh.at[pl.ds(s * ROWS_S, ROWS_S)],
                            out_hbm.at[r, c, pl.ds(s * ROWS_S, ROWS_S)])
            if r + 1 < R:
                pltpu.sync_copy(src_hbm.at[r + 1, wid], src_v[(r + 1) % 2])
                pltpu.sync_copy(dst_hbm.at[r + 1, wid], dst_v[(r + 1) % 2])
                plsc.subcore_barrier()

    return sc_agg


@functools.lru_cache(maxsize=None)
def _get_sc_agg():
    return _make_sc_agg()


def _cnt_body(d_ref, inv_ref, acc_ref):
    i = pl.program_id(0)

    @pl.when(i % NBE == 0)
    def _():
        acc_ref[...] = jnp.zeros((NPAD // H, H), jnp.float32)

    d = d_ref[0, :, :]
    d = jnp.reshape(d, (BE, 1))
    q = lax.shift_right_logical(d, 7)
    m = lax.bitwise_and(d, 127)
    ohq = (q == lax.broadcasted_iota(jnp.int32, (BE, NPAD // H), 1))
    ohm = (m == lax.broadcasted_iota(jnp.int32, (BE, H), 1))
    acc_ref[...] += lax.dot_general(
        ohq.astype(jnp.float32), ohm.astype(jnp.float32),
        (((0,), (0,)), ((), ())), preferred_element_type=jnp.float32)

    @pl.when(i % NBE == NBE - 1)
    def _():
        inv_ref[...] = (1.0 / jnp.maximum(acc_ref[...], 1.0))[None]


def _cnt(dst_rs):
    return pl.pallas_call(
        _cnt_body,
        grid=(R * NBE,),
        in_specs=[pl.BlockSpec((1, 1, BE), lambda i: (i, 0, 0))],
        out_specs=pl.BlockSpec((1, NPAD // H, H), lambda i: (i // NBE, 0, 0)),
        out_shape=jax.ShapeDtypeStruct((R, NPAD // H, H), jnp.float32),
        scratch_shapes=[pltpu.VMEM((NPAD // H, H), jnp.float32)],
    )(dst_rs)


def _y0_body(x_ref, w_ref, y_ref):
    x = x_ref[...]
    for r in range(R):
        y_ref[r] = jnp.dot(x, w_ref[r], preferred_element_type=jnp.float32)


def _y0(xp, wr0):
    f = xp.shape[1]
    return pl.pallas_call(
        _y0_body,
        grid=(NPAD // BN,),
        in_specs=[
            pl.BlockSpec((BN, f), lambda i: (i, 0)),
            pl.BlockSpec((R, f, H), lambda i: (0, 0, 0)),
        ],
        out_specs=pl.BlockSpec((R, BN, H), lambda i: (0, i, 0)),
        out_shape=jax.ShapeDtypeStruct((R, NPAD, H), jnp.float32),
    )(xp, wr0)


def _dense_mid_body(x_ref, p_ref, ic_ref, rt_ref, b_ref, wn_ref,
                    h_ref, y_ref):
    acc = jnp.dot(x_ref[...], rt_ref[...], preferred_element_type=jnp.float32)
    acc = acc + b_ref[...]
    ic = ic_ref[...]
    s_prev = jnp.zeros((BN, H), jnp.float32)
    for r in range(R):
        s_cur = p_ref[r, 0] + p_ref[r, 1]
        acc = acc + (s_cur - s_prev) * ic[:, r][:, None]
        s_prev = s_cur
    h = jnp.maximum(acc, 0.0)
    h_ref[...] = h
    for r in range(R):
        y_ref[r] = jnp.dot(h, wn_ref[r], preferred_element_type=jnp.float32)


def _dense_mid(x, parts, invc, root, b, wnext):
    f = x.shape[1]
    return pl.pallas_call(
        _dense_mid_body,
        grid=(NPAD // BN,),
        in_specs=[
            pl.BlockSpec((BN, f), lambda i: (i, 0)),
            pl.BlockSpec((R, NC, BN, H), lambda i: (0, 0, i, 0)),
            pl.BlockSpec((BN, 8), lambda i: (i, 0)),
            pl.BlockSpec((f, H), lambda i: (0, 0)),
            pl.BlockSpec((1, H), lambda i: (0, 0)),
            pl.BlockSpec((R, H, H), lambda i: (0, 0, 0)),
        ],
        out_specs=[
            pl.BlockSpec((BN, H), lambda i: (i, 0)),
            pl.BlockSpec((R, BN, H), lambda i: (0, i, 0)),
        ],
        out_shape=[
            jax.ShapeDtypeStruct((NPAD, H), jnp.float32),
            jax.ShapeDtypeStruct((R, NPAD, H), jnp.float32),
        ],
    )(x, parts, invc, root, b, wnext)


def _dense_last_body(x_ref, p_ref, ic_ref, rt_ref, b_ref, h_ref):
    acc = jnp.dot(x_ref[...], rt_ref[...], preferred_element_type=jnp.float32)
    acc = acc + b_ref[...]
    ic = ic_ref[...]
    s_prev = jnp.zeros((BN, H), jnp.float32)
    for r in range(R):
        s_cur = p_ref[r, 0] + p_ref[r, 1]
        acc = acc + (s_cur - s_prev) * ic[:, r][:, None]
        s_prev = s_cur
    h_ref[...] = jnp.maximum(acc, 0.0)


def _dense_last(x, parts, invc, root, b):
    return pl.pallas_call(
        _dense_last_body,
        grid=(NPAD // BN,),
        in_specs=[
            pl.BlockSpec((BN, H), lambda i: (i, 0)),
            pl.BlockSpec((R, NC, BN, H), lambda i: (0, 0, i, 0)),
            pl.BlockSpec((BN, 8), lambda i: (i, 0)),
            pl.BlockSpec((H, H), lambda i: (0, 0)),
            pl.BlockSpec((1, H), lambda i: (0, 0)),
        ],
        out_specs=pl.BlockSpec((BN, H), lambda i: (i, 0)),
        out_shape=jax.ShapeDtypeStruct((NPAD, H), jnp.float32),
    )(x, parts, invc, root, b)


def _head_body(h_ref, bt_ref, w1_ref, b1_ref, w2_ref, b2_ref, w3_ref, b3_ref,
               out_ref, acc_ref):
    i = pl.program_id(0)

    @pl.when(i == 0)
    def _():
        acc_ref[...] = jnp.zeros((G, H), jnp.float32)

    bids = bt_ref[...]
    onehot = (bids == lax.broadcasted_iota(jnp.int32, (BN, G), 1))
    onehot = onehot.astype(jnp.float32)
    acc_ref[...] += lax.dot_general(
        onehot, h_ref[...], (((0,), (0,)), ((), ())),
        preferred_element_type=jnp.float32)

    @pl.when(i == pl.num_programs(0) - 1)
    def _():
        ro = acc_ref[...]
        z = jnp.maximum(jnp.dot(ro, w1_ref[...],
                                preferred_element_type=jnp.float32)
                        + b1_ref[...], 0.0)
        z = jnp.maximum(jnp.dot(z, w2_ref[...],
                                preferred_element_type=jnp.float32)
                        + b2_ref[...], 0.0)
        out_ref[...] = jnp.dot(z, w3_ref[...],
                               preferred_element_type=jnp.float32) + b3_ref[...]


def _head(h3, batch_p, w1, b1, w2, b2, w3, b3):
    return pl.pallas_call(
        _head_body,
        grid=(NPAD // BN,),
        in_specs=[
            pl.BlockSpec((BN, H), lambda i: (i, 0)),
            pl.BlockSpec((BN, 1), lambda i: (i, 0)),
            pl.BlockSpec((H, H), lambda i: (0, 0)),
            pl.BlockSpec((1, H), lambda i: (0, 0)),
            pl.BlockSpec((H, H), lambda i: (0, 0)),
            pl.BlockSpec((1, H), lambda i: (0, 0)),
            pl.BlockSpec((H, 1), lambda i: (0, 0)),
            pl.BlockSpec((1, 1), lambda i: (0, 0)),
        ],
        out_specs=pl.BlockSpec((G, 1), lambda i: (0, 0)),
        out_shape=jax.ShapeDtypeStruct((G, 1), jnp.float32),
        scratch_shapes=[pltpu.VMEM((G, H), jnp.float32)],
    )(h3, batch_p, w1, b1, w2, b2, w3, b3)


def kernel(X, edge_index1, edge_index2, edge_index3, edge_index4, edge_index5,
           batch, Wr0, root0, b0, Wr1, root1, b1, Wr2, root2, b2,
           W1, bl1, W2, bl2, W3, bl3):
    f32 = jnp.float32
    xp = jnp.concatenate([X, jnp.zeros((NPAD - N, FIN), f32)], axis=0)

    eis = jnp.stack([edge_index1, edge_index2, edge_index3,
                     edge_index4, edge_index5])
    src = jnp.concatenate(
        [eis[:, 0, :], jnp.zeros((R, EPAD - E), jnp.int32)], axis=1)
    dst = jnp.concatenate(
        [eis[:, 1, :], jnp.full((R, EPAD - E), TRASH, jnp.int32)], axis=1)
    dst_rs = dst.reshape(R * NBE, 1, BE)
    src = src.reshape(R, NW, NCH, CH)
    dstw = dst.reshape(R, NW, NCH, CH)

    batch_p = jnp.concatenate(
        [batch, jnp.full((NPAD - N,), G, jnp.int32)]).reshape(NPAD, 1)
    zer = jnp.zeros((ZROWS, H), f32)

    inv = _cnt(dst_rs)
    invc = jnp.concatenate(
        [inv.reshape(R, NPAD).T, jnp.ones((NPAD, 3), f32)], axis=1)

    sc_agg = _get_sc_agg()
    y0 = _y0(xp, Wr0)
    parts0 = sc_agg(y0[0], y0[1], y0[2], y0[3], y0[4], src, dstw, zer)
    h1, y1 = _dense_mid(xp, parts0, invc, root0, b0.reshape(1, H), Wr1)
    parts1 = sc_agg(y1[0], y1[1], y1[2], y1[3], y1[4], src, dstw, zer)
    h2, y2 = _dense_mid(h1, parts1, invc, root1, b1.reshape(1, H), Wr2)
    parts2 = sc_agg(y2[0], y2[1], y2[2], y2[3], y2[4], src, dstw, zer)
    h3 = _dense_last(h2, parts2, invc, root2, b2.reshape(1, H))
    return _head(h3, batch_p, W1, bl1.reshape(1, H), W2, bl2.reshape(1, H),
                 W3, bl3.reshape(1, 1))
